# Initial kernel scaffold; baseline (speedup 1.0000x reference)
import jax
import jax.numpy as jnp
from jax.experimental import pallas as pl


def kernel(x, edge_index, W, bn_gamma, bn_beta):
    raise NotImplementedError("write your pallas kernel here")


# speedup vs baseline: 13.6799x; 13.6799x over previous
import functools

import jax
import jax.numpy as jnp
from jax import lax
from jax.experimental import pallas as pl
from jax.experimental.pallas import tpu as pltpu
from jax.experimental.pallas import tpu_sc as plsc

N = 10000
E = 160000
D = 256
H = 128
NC, NS = 2, 16
NPAD = 10240
EB = 80
ERows = E // EB
RPT = ERows // NS
RB = 1000
EPS_BN = 1e-5

_sc_mesh = plsc.VectorSubcoreMesh(core_axis_name="c", subcore_axis_name="s")


@functools.partial(
    pl.kernel,
    out_type=jax.ShapeDtypeStruct((NC * NPAD,), jnp.float32),
    mesh=_sc_mesh,
    scratch_types=[
        pltpu.VMEM((RPT, EB), jnp.int32),
        pltpu.VMEM((EB,), jnp.float32),
        pltpu.VMEM((640,), jnp.float32),
        pltpu.VMEM_SHARED((NPAD,), jnp.float32),
    ],
)
def _deg_kernel(dst3_hbm, out_hbm, idx_v, ones_v, zeros_v, deg_sh):
    c = lax.axis_index("c")
    s = lax.axis_index("s")

    def fill_ones(i, carry):
        ones_v[pl.ds(i * 16, 16)] = jnp.full((16,), 1.0, jnp.float32)
        return carry

    lax.fori_loop(0, EB // 16, fill_ones, 0)

    def fill_zeros(i, carry):
        zeros_v[pl.ds(i * 16, 16)] = jnp.zeros((16,), jnp.float32)
        return carry

    lax.fori_loop(0, 640 // 16, fill_zeros, 0)
    pltpu.sync_copy(zeros_v, deg_sh.at[pl.ds(s * 640, 640)])
    plsc.subcore_barrier()

    pltpu.sync_copy(dst3_hbm.at[s], idx_v)

    def scat(b, carry):
        pltpu.sync_copy(ones_v, deg_sh.at[idx_v.at[b]], add=True)
        return carry

    lax.fori_loop(0, RPT, scat, 0)
    plsc.subcore_barrier()
    pltpu.sync_copy(deg_sh.at[pl.ds(s * 640, 640)],
                    out_hbm.at[pl.ds(c * NPAD + s * 640, 640)])


@functools.partial(
    pl.kernel,
    out_type=jax.ShapeDtypeStruct((NC, NPAD, H), jnp.float32),
    mesh=_sc_mesh,
    scratch_types=[
        pltpu.VMEM((RPT, EB), jnp.int32),
        pltpu.VMEM((RPT, EB), jnp.int32),
        pltpu.VMEM((EB, H), jnp.float32),
        pltpu.VMEM((8, H), jnp.float32),
        pltpu.VMEM_SHARED((NPAD, H), jnp.float32),
        pltpu.SemaphoreType.DMA,
    ],
)
def _agg_kernel(src3_hbm, dst3_hbm, xt2_hbm, out_hbm,
                sidx_v, didx_v, rows_v, zeros_v, acc_sh, sem):
    c = lax.axis_index("c")
    s = lax.axis_index("s")

    def fill_zeros(i, carry):
        zeros_v[i // 8, pl.ds((i % 8) * 16, 16)] = jnp.zeros((16,), jnp.float32)
        return carry

    lax.fori_loop(0, 8 * H // 16, fill_zeros, 0)

    def zero_acc(j, carry):
        pltpu.sync_copy(zeros_v, acc_sh.at[pl.ds(s * 640 + j * 8, 8)])
        return carry

    lax.fori_loop(0, 80, zero_acc, 0)
    plsc.subcore_barrier()

    pltpu.sync_copy(src3_hbm.at[s], sidx_v)
    pltpu.sync_copy(dst3_hbm.at[s], didx_v)

    def scale_idx(i, carry):
        r = i // (EB // 16)
        l = i % (EB // 16)
        v = sidx_v[r, pl.ds(l * 16, 16)]
        sidx_v[r, pl.ds(l * 16, 16)] = v * 2 + c
        return carry

    lax.fori_loop(0, RPT * (EB // 16), scale_idx, 0)

    def batch(b, carry):
        pltpu.async_copy(xt2_hbm.at[sidx_v.at[b]], rows_v, sem).wait()
        pltpu.sync_copy(rows_v, acc_sh.at[didx_v.at[b]], add=True)
        return carry

    lax.fori_loop(0, RPT, batch, 0)
    plsc.subcore_barrier()
    pltpu.sync_copy(acc_sh.at[pl.ds(s * 640, 640)],
                    out_hbm.at[c, pl.ds(s * 640, 640)])


def _prep_body(degp_ref, x_ref, xt_ref, dis_ref, invd_ref):
    d = (degp_ref[:, 0:1] + degp_ref[:, 1:2]) * 0.5 + 1.0
    dis = lax.rsqrt(d)
    invd = 1.0 / d
    xt_ref[...] = x_ref[...] * dis
    dis_ref[...] = dis
    invd_ref[...] = invd


def _prep(degp_t, x):
    nb = N // RB
    return pl.pallas_call(
        _prep_body,
        grid=(nb,),
        in_specs=[
            pl.BlockSpec((RB, NC), lambda i: (i, 0)),
            pl.BlockSpec((RB, D), lambda i: (i, 0)),
        ],
        out_specs=[
            pl.BlockSpec((RB, D), lambda i: (i, 0)),
            pl.BlockSpec((RB, 1), lambda i: (i, 0)),
            pl.BlockSpec((RB, 1), lambda i: (i, 0)),
        ],
        out_shape=[
            jax.ShapeDtypeStruct((N, D), jnp.float32),
            jax.ShapeDtypeStruct((N, 1), jnp.float32),
            jax.ShapeDtypeStruct((N, 1), jnp.float32),
        ],
    )(degp_t, x)


def _mm_body(al_ref, ar_ref, x_ref, dis_ref, invd_ref, w_ref,
             p_ref, s1_ref, s2_ref):
    i = pl.program_id(0)
    a = jnp.concatenate([al_ref[0], ar_ref[0]], axis=1)
    b = a * dis_ref[...] + x_ref[...] * invd_ref[...]
    p = jnp.dot(b, w_ref[...], preferred_element_type=jnp.float32)
    p_ref[...] = p
    s1 = jnp.sum(p, axis=0, keepdims=True)
    s2 = jnp.sum(p * p, axis=0, keepdims=True)

    @pl.when(i == 0)
    def _():
        s1_ref[...] = s1
        s2_ref[...] = s2

    @pl.when(i != 0)
    def _():
        s1_ref[...] += s1
        s2_ref[...] += s2


def _mm(afull, x, dis2, invd2, w):
    nb = N // RB
    return pl.pallas_call(
        _mm_body,
        grid=(nb,),
        in_specs=[
            pl.BlockSpec((1, RB, H), lambda i: (0, i, 0)),
            pl.BlockSpec((1, RB, H), lambda i: (1, i, 0)),
            pl.BlockSpec((RB, D), lambda i: (i, 0)),
            pl.BlockSpec((RB, 1), lambda i: (i, 0)),
            pl.BlockSpec((RB, 1), lambda i: (i, 0)),
            pl.BlockSpec((D, D), lambda i: (0, 0)),
        ],
        out_specs=[
            pl.BlockSpec((RB, D), lambda i: (i, 0)),
            pl.BlockSpec((1, D), lambda i: (0, 0)),
            pl.BlockSpec((1, D), lambda i: (0, 0)),
        ],
        out_shape=[
            jax.ShapeDtypeStruct((N, D), jnp.float32),
            jax.ShapeDtypeStruct((1, D), jnp.float32),
            jax.ShapeDtypeStruct((1, D), jnp.float32),
        ],
    )(afull, afull, x, dis2, invd2, w)


def _bn_body(p_ref, s1_ref, s2_ref, g_ref, b_ref, o_ref):
    mean = s1_ref[...] * (1.0 / N)
    var = s2_ref[...] * (1.0 / N) - mean * mean
    scale = g_ref[...] * lax.rsqrt(var + EPS_BN)
    shift = b_ref[...] - mean * scale
    o_ref[...] = jnp.maximum(p_ref[...] * scale + shift, 0.0)


def _bn(p, s1, s2, gamma, beta):
    nb = N // RB
    return pl.pallas_call(
        _bn_body,
        grid=(nb,),
        in_specs=[
            pl.BlockSpec((RB, D), lambda i: (i, 0)),
            pl.BlockSpec((1, D), lambda i: (0, 0)),
            pl.BlockSpec((1, D), lambda i: (0, 0)),
            pl.BlockSpec((1, D), lambda i: (0, 0)),
            pl.BlockSpec((1, D), lambda i: (0, 0)),
        ],
        out_specs=pl.BlockSpec((RB, D), lambda i: (i, 0)),
        out_shape=jax.ShapeDtypeStruct((N, D), jnp.float32),
    )(p, s1, s2, gamma, beta)


def kernel(x, edge_index, W, bn_gamma, bn_beta):
    ei = edge_index.astype(jnp.int32)
    src3 = ei[0].reshape(NS, RPT, EB)
    dst3 = ei[1].reshape(NS, RPT, EB)

    degp = _deg_kernel(dst3).reshape(NC, NPAD)
    degp_t = jnp.transpose(degp)[:N]
    xt, dis2, invd2 = _prep(degp_t, x)
    xt2 = xt.reshape(2 * N, H)
    afull = _agg_kernel(src3, dst3, xt2)
    p, s1, s2 = _mm(afull, x, dis2, invd2, W)
    return _bn(p, s1, s2, bn_gamma.reshape(1, D), bn_beta.reshape(1, D))


# speedup vs baseline: 19.0364x; 1.3916x over previous
import functools

import jax
import jax.numpy as jnp
from jax import lax
from jax.experimental import pallas as pl
from jax.experimental.pallas import tpu as pltpu
from jax.experimental.pallas import tpu_sc as plsc

N = 10000
E = 160000
D = 256
H = 128
NC, NS = 2, 16
NPAD = 10240
EB = 80
ERows = E // EB
RPT = ERows // NS
CH = 25
NCH = RPT // CH
RB = 1000
NBLK = N // RB
EPS_BN = 1e-5

_sc_mesh = plsc.VectorSubcoreMesh(core_axis_name="c", subcore_axis_name="s")


@functools.partial(
    pl.kernel,
    out_type=jax.ShapeDtypeStruct((NC * NPAD,), jnp.float32),
    mesh=_sc_mesh,
    scratch_types=[
        pltpu.VMEM((NCH, CH, EB), jnp.int32),
        pltpu.VMEM((EB,), jnp.float32),
        pltpu.VMEM((640,), jnp.float32),
        pltpu.VMEM_SHARED((NPAD,), jnp.float32),
    ],
)
def _deg_kernel(dst4_hbm, out_hbm, idx_v, ones_v, zeros_v, deg_sh):
    c = lax.axis_index("c")
    s = lax.axis_index("s")

    def fill_ones(i, carry):
        ones_v[pl.ds(i * 16, 16)] = jnp.full((16,), 1.0, jnp.float32)
        return carry

    lax.fori_loop(0, EB // 16, fill_ones, 0)

    def fill_zeros(i, carry):
        zeros_v[pl.ds(i * 16, 16)] = jnp.zeros((16,), jnp.float32)
        return carry

    lax.fori_loop(0, 640 // 16, fill_zeros, 0)
    pltpu.sync_copy(zeros_v, deg_sh.at[pl.ds(s * 640, 640)])
    plsc.subcore_barrier()

    pltpu.sync_copy(dst4_hbm.at[s], idx_v)

    def scat(b, carry):
        pltpu.sync_copy(ones_v, deg_sh.at[idx_v.at[b // CH, b % CH]],
                        add=True)
        return carry

    lax.fori_loop(0, RPT, scat, 0)
    plsc.subcore_barrier()
    pltpu.sync_copy(deg_sh.at[pl.ds(s * 640, 640)],
                    out_hbm.at[pl.ds(c * NPAD + s * 640, 640)])


@functools.partial(
    pl.kernel,
    out_type=jax.ShapeDtypeStruct((NC, NPAD, H), jnp.float32),
    mesh=_sc_mesh,
    scratch_types=[
        pltpu.VMEM((CH, EB), jnp.int32),
        pltpu.VMEM((CH, EB), jnp.int32),
        pltpu.VMEM((2, EB, H), jnp.float32),
        pltpu.VMEM((8, H), jnp.float32),
        pltpu.VMEM_SHARED((NPAD, H), jnp.float32),
        pltpu.SemaphoreType.DMA,
        pltpu.SemaphoreType.DMA,
    ],
)
def _agg_kernel(src4_hbm, dst4_hbm, xt2_hbm, out_hbm,
                sidx_v, didx_v, rows_v, zeros_v, acc_sh, gsem, ssem):
    c = lax.axis_index("c")
    s = lax.axis_index("s")

    def fill_zeros(i, carry):
        zeros_v[i // 8, pl.ds((i % 8) * 16, 16)] = jnp.zeros((16,), jnp.float32)
        return carry

    lax.fori_loop(0, 8 * H // 16, fill_zeros, 0)

    def zero_acc(j, carry):
        pltpu.sync_copy(zeros_v, acc_sh.at[pl.ds(s * 640 + j * 8, 8)])
        return carry

    lax.fori_loop(0, 80, zero_acc, 0)
    plsc.subcore_barrier()

    def drain_gather(par):
        pltpu.make_async_copy(xt2_hbm.at[pl.ds(0, EB)], rows_v.at[par],
                              gsem).wait()

    def drain_scatter():
        pltpu.make_async_copy(rows_v.at[0], acc_sh.at[pl.ds(0, EB)],
                              ssem).wait()

    def chunk(ch, carry):
        pltpu.sync_copy(src4_hbm.at[s, ch], sidx_v)
        pltpu.sync_copy(dst4_hbm.at[s, ch], didx_v)

        def scale_idx(i, carry2):
            r = i // (EB // 16)
            l = i % (EB // 16)
            v = sidx_v[r, pl.ds(l * 16, 16)]
            sidx_v[r, pl.ds(l * 16, 16)] = v * 2 + c
            return carry2

        lax.fori_loop(0, CH * (EB // 16), scale_idx, 0)

        pltpu.async_copy(xt2_hbm.at[sidx_v.at[0]], rows_v.at[0], gsem)

        def batch(b, carry2):
            par = lax.rem(b, 2)
            nxt = 1 - par

            @pl.when(b + 1 < CH)
            def _():
                @pl.when(b >= 1)
                def _():
                    drain_scatter()
                pltpu.async_copy(xt2_hbm.at[sidx_v.at[b + 1]],
                                 rows_v.at[nxt], gsem)

            drain_gather(par)
            pltpu.async_copy(rows_v.at[par], acc_sh.at[didx_v.at[b]],
                             ssem, add=True)
            return carry2

        lax.fori_loop(0, CH, batch, 0)
        drain_scatter()
        drain_scatter()
        return carry

    lax.fori_loop(0, NCH, chunk, 0)
    plsc.subcore_barrier()
    pltpu.sync_copy(acc_sh.at[pl.ds(s * 640, 640)],
                    out_hbm.at[c, pl.ds(s * 640, 640)])


def _prep_body(degp_ref, x_ref, xt_ref, dis_ref, invd_ref):
    d = (degp_ref[:, 0:1] + degp_ref[:, 1:2]) * 0.5 + 1.0
    dis = lax.rsqrt(d)
    invd = 1.0 / d
    xt_ref[...] = x_ref[...] * dis
    dis_ref[...] = dis
    invd_ref[...] = invd


def _prep(degp_t, x):
    nb = N // RB
    return pl.pallas_call(
        _prep_body,
        grid=(nb,),
        in_specs=[
            pl.BlockSpec((RB, NC), lambda i: (i, 0)),
            pl.BlockSpec((RB, D), lambda i: (i, 0)),
        ],
        out_specs=[
            pl.BlockSpec((RB, D), lambda i: (i, 0)),
            pl.BlockSpec((RB, 1), lambda i: (i, 0)),
            pl.BlockSpec((RB, 1), lambda i: (i, 0)),
        ],
        out_shape=[
            jax.ShapeDtypeStruct((N, D), jnp.float32),
            jax.ShapeDtypeStruct((N, 1), jnp.float32),
            jax.ShapeDtypeStruct((N, 1), jnp.float32),
        ],
    )(degp_t, x)


def _mmbn_body(al_ref, ar_ref, x_ref, dis_ref, invd_ref, w_ref, g_ref, b_ref,
               o_ref, p_scr, s1_scr, s2_scr):
    ph = pl.program_id(0)
    i = pl.program_id(1)

    @pl.when(ph == 0)
    def _():
        a = jnp.concatenate([al_ref[0], ar_ref[0]], axis=1)
        bm = a * dis_ref[...] + x_ref[...] * invd_ref[...]
        p = jnp.dot(bm, w_ref[...], preferred_element_type=jnp.float32)
        p_scr[pl.ds(i * RB, RB), :] = p
        s1 = jnp.sum(p, axis=0, keepdims=True)
        s2 = jnp.sum(p * p, axis=0, keepdims=True)

        @pl.when(i == 0)
        def _():
            s1_scr[...] = s1
            s2_scr[...] = s2

        @pl.when(i != 0)
        def _():
            s1_scr[...] += s1
            s2_scr[...] += s2

    @pl.when(ph == 1)
    def _():
        mean = s1_scr[...] * (1.0 / N)
        var = s2_scr[...] * (1.0 / N) - mean * mean
        scale = g_ref[...] * lax.rsqrt(var + EPS_BN)
        shift = b_ref[...] - mean * scale
        p = p_scr[pl.ds(i * RB, RB), :]
        o_ref[...] = jnp.maximum(p * scale + shift, 0.0)


def _mmbn(afull, x, dis2, invd2, w, gamma, beta):
    def only0(f):
        return lambda ph, i: f(jnp.where(ph == 0, i, NBLK - 1))

    return pl.pallas_call(
        _mmbn_body,
        grid=(2, NBLK),
        in_specs=[
            pl.BlockSpec((1, RB, H), only0(lambda i: (0, i, 0))),
            pl.BlockSpec((1, RB, H), only0(lambda i: (1, i, 0))),
            pl.BlockSpec((RB, D), only0(lambda i: (i, 0))),
            pl.BlockSpec((RB, 1), only0(lambda i: (i, 0))),
            pl.BlockSpec((RB, 1), only0(lambda i: (i, 0))),
            pl.BlockSpec((D, D), lambda ph, i: (0, 0)),
            pl.BlockSpec((1, D), lambda ph, i: (0, 0)),
            pl.BlockSpec((1, D), lambda ph, i: (0, 0)),
        ],
        out_specs=pl.BlockSpec((RB, D), lambda ph, i: (i, 0)),
        out_shape=jax.ShapeDtypeStruct((N, D), jnp.float32),
        scratch_shapes=[
            pltpu.VMEM((N, D), jnp.float32),
            pltpu.VMEM((1, D), jnp.float32),
            pltpu.VMEM((1, D), jnp.float32),
        ],
    )(afull, afull, x, dis2, invd2, w, gamma, beta)


def kernel(x, edge_index, W, bn_gamma, bn_beta):
    ei = edge_index.astype(jnp.int32)
    src4 = ei[0].reshape(NS, NCH, CH, EB)
    dst4 = ei[1].reshape(NS, NCH, CH, EB)

    degp = _deg_kernel(dst4).reshape(NC, NPAD)
    degp_t = jnp.transpose(degp)[:N]
    xt, dis2, invd2 = _prep(degp_t, x)
    xt2 = xt.reshape(2 * N, H)
    afull = _agg_kernel(src4, dst4, xt2)
    return _mmbn(afull, x, dis2, invd2, W,
                 bn_gamma.reshape(1, D), bn_beta.reshape(1, D))


# speedup vs baseline: 22.2521x; 1.1689x over previous
import functools

import jax
import jax.numpy as jnp
from jax import lax
from jax.experimental import pallas as pl
from jax.experimental.pallas import tpu as pltpu
from jax.experimental.pallas import tpu_sc as plsc

N = 10000
E = 160000
D = 256
H = 128
NC, NS = 2, 16
NPAD = 10240
EB = 80
ERows = E // EB
RPT = ERows // NS
CH = 25
NCH = RPT // CH
RB = 1000
NBLK = N // RB
EPS_BN = 1e-5

_sc_mesh = plsc.VectorSubcoreMesh(core_axis_name="c", subcore_axis_name="s")


@functools.partial(
    pl.kernel,
    out_type=jax.ShapeDtypeStruct((NC * NPAD,), jnp.float32),
    mesh=_sc_mesh,
    scratch_types=[
        pltpu.VMEM((NCH, CH, EB), jnp.int32),
        pltpu.VMEM((EB,), jnp.float32),
        pltpu.VMEM((640,), jnp.float32),
        pltpu.VMEM_SHARED((NPAD,), jnp.float32),
        pltpu.SemaphoreType.DMA,
    ],
)
def _deg_kernel(dst4_hbm, out_hbm, idx_v, ones_v, zeros_v, deg_sh, sem):
    c = lax.axis_index("c")
    s = lax.axis_index("s")

    def fill_ones(i, carry):
        ones_v[pl.ds(i * 16, 16)] = jnp.full((16,), 1.0, jnp.float32)
        return carry

    lax.fori_loop(0, EB // 16, fill_ones, 0)

    def fill_zeros(i, carry):
        zeros_v[pl.ds(i * 16, 16)] = jnp.zeros((16,), jnp.float32)
        return carry

    lax.fori_loop(0, 640 // 16, fill_zeros, 0)
    pltpu.sync_copy(zeros_v, deg_sh.at[pl.ds(s * 640, 640)])
    plsc.subcore_barrier()

    pltpu.sync_copy(dst4_hbm.at[s], idx_v)

    def scat(b, carry):
        pltpu.async_copy(ones_v, deg_sh.at[idx_v.at[b // CH, b % CH]],
                         sem, add=True)
        return carry

    lax.fori_loop(0, RPT, scat, 0)

    def drain(b, carry):
        pltpu.make_async_copy(ones_v, deg_sh.at[pl.ds(0, EB)], sem).wait()
        return carry

    lax.fori_loop(0, RPT, drain, 0)
    plsc.subcore_barrier()
    pltpu.sync_copy(deg_sh.at[pl.ds(s * 640, 640)],
                    out_hbm.at[pl.ds(c * NPAD + s * 640, 640)])


@functools.partial(
    pl.kernel,
    out_type=jax.ShapeDtypeStruct((NC, NPAD, H), jnp.float32),
    mesh=_sc_mesh,
    scratch_types=[
        pltpu.VMEM((CH, EB), jnp.int32),
        pltpu.VMEM((CH, EB), jnp.int32),
        pltpu.VMEM((3, EB, H), jnp.float32),
        pltpu.VMEM_SHARED((NPAD, H), jnp.float32),
        pltpu.SemaphoreType.DMA,
        pltpu.SemaphoreType.DMA,
    ],
)
def _agg_kernel(src4_hbm, dst4_hbm, xt2_hbm, out_hbm,
                sidx_v, didx_v, rows_v, acc_sh, gsem, ssem):
    c = lax.axis_index("c")
    s = lax.axis_index("s")

    def fill_zeros(i, carry):
        rows_v[0, i // 8, pl.ds((i % 8) * 16, 16)] = jnp.zeros((16,),
                                                              jnp.float32)
        return carry

    lax.fori_loop(0, EB * H // 16, fill_zeros, 0)

    def zero_acc(j, carry):
        pltpu.sync_copy(rows_v.at[0], acc_sh.at[pl.ds(s * 640 + j * EB, EB)])
        return carry

    lax.fori_loop(0, 8, zero_acc, 0)
    plsc.subcore_barrier()

    def drain_gather(par):
        pltpu.make_async_copy(xt2_hbm.at[pl.ds(0, EB)], rows_v.at[par],
                              gsem).wait()

    def drain_scatter():
        pltpu.make_async_copy(rows_v.at[0], acc_sh.at[pl.ds(0, EB)],
                              ssem).wait()

    def chunk(ch, carry):
        pltpu.sync_copy(src4_hbm.at[s, ch], sidx_v)
        pltpu.sync_copy(dst4_hbm.at[s, ch], didx_v)

        def scale_idx(i, carry2):
            r = i // (EB // 16)
            l = i % (EB // 16)
            v = sidx_v[r, pl.ds(l * 16, 16)]
            sidx_v[r, pl.ds(l * 16, 16)] = v * 2 + c
            return carry2

        lax.fori_loop(0, CH * (EB // 16), scale_idx, 0)

        pltpu.async_copy(xt2_hbm.at[sidx_v.at[0]], rows_v.at[0], gsem)
        pltpu.async_copy(xt2_hbm.at[sidx_v.at[1]], rows_v.at[1], gsem)

        def batch(b, carry2):
            cur = lax.rem(b, 3)

            @pl.when(b + 2 < CH)
            def _():
                @pl.when(b >= 1)
                def _():
                    drain_scatter()
                pltpu.async_copy(xt2_hbm.at[sidx_v.at[b + 2]],
                                 rows_v.at[lax.rem(b + 2, 3)], gsem)

            drain_gather(cur)
            pltpu.async_copy(rows_v.at[cur], acc_sh.at[didx_v.at[b]],
                             ssem, add=True)
            return carry2

        lax.fori_loop(0, CH, batch, 0)
        drain_scatter()
        drain_scatter()
        drain_scatter()
        return carry

    lax.fori_loop(0, NCH, chunk, 0)
    plsc.subcore_barrier()
    pltpu.sync_copy(acc_sh.at[pl.ds(s * 640, 640)],
                    out_hbm.at[c, pl.ds(s * 640, 640)])


def _prep_body(degp_ref, x_ref, xt_ref, dis_ref, invd_ref):
    d = (degp_ref[:, 0:1] + degp_ref[:, 1:2]) * 0.5 + 1.0
    dis = lax.rsqrt(d)
    invd = 1.0 / d
    xt_ref[...] = x_ref[...] * dis
    dis_ref[...] = dis
    invd_ref[...] = invd


def _prep(degp_t, x):
    nb = N // RB
    return pl.pallas_call(
        _prep_body,
        grid=(nb,),
        in_specs=[
            pl.BlockSpec((RB, NC), lambda i: (i, 0)),
            pl.BlockSpec((RB, D), lambda i: (i, 0)),
        ],
        out_specs=[
            pl.BlockSpec((RB, D), lambda i: (i, 0)),
            pl.BlockSpec((RB, 1), lambda i: (i, 0)),
            pl.BlockSpec((RB, 1), lambda i: (i, 0)),
        ],
        out_shape=[
            jax.ShapeDtypeStruct((N, D), jnp.float32),
            jax.ShapeDtypeStruct((N, 1), jnp.float32),
            jax.ShapeDtypeStruct((N, 1), jnp.float32),
        ],
    )(degp_t, x)


def _mmbn_body(al_ref, ar_ref, x_ref, dis_ref, invd_ref, w_ref, g_ref, b_ref,
               o_ref, p_scr, s1_scr, s2_scr):
    ph = pl.program_id(0)
    i = pl.program_id(1)

    @pl.when(ph == 0)
    def _():
        a = jnp.concatenate([al_ref[0], ar_ref[0]], axis=1)
        bm = a * dis_ref[...] + x_ref[...] * invd_ref[...]
        p = jnp.dot(bm, w_ref[...], preferred_element_type=jnp.float32)
        p_scr[pl.ds(i * RB, RB), :] = p
        s1 = jnp.sum(p, axis=0, keepdims=True)
        s2 = jnp.sum(p * p, axis=0, keepdims=True)

        @pl.when(i == 0)
        def _():
            s1_scr[...] = s1
            s2_scr[...] = s2

        @pl.when(i != 0)
        def _():
            s1_scr[...] += s1
            s2_scr[...] += s2

    @pl.when(ph == 1)
    def _():
        mean = s1_scr[...] * (1.0 / N)
        var = s2_scr[...] * (1.0 / N) - mean * mean
        scale = g_ref[...] * lax.rsqrt(var + EPS_BN)
        shift = b_ref[...] - mean * scale
        p = p_scr[pl.ds(i * RB, RB), :]
        o_ref[...] = jnp.maximum(p * scale + shift, 0.0)


def _mmbn(afull, x, dis2, invd2, w, gamma, beta):
    def only0(f):
        return lambda ph, i: f(jnp.where(ph == 0, i, NBLK - 1))

    return pl.pallas_call(
        _mmbn_body,
        grid=(2, NBLK),
        in_specs=[
            pl.BlockSpec((1, RB, H), only0(lambda i: (0, i, 0))),
            pl.BlockSpec((1, RB, H), only0(lambda i: (1, i, 0))),
            pl.BlockSpec((RB, D), only0(lambda i: (i, 0))),
            pl.BlockSpec((RB, 1), only0(lambda i: (i, 0))),
            pl.BlockSpec((RB, 1), only0(lambda i: (i, 0))),
            pl.BlockSpec((D, D), lambda ph, i: (0, 0)),
            pl.BlockSpec((1, D), lambda ph, i: (0, 0)),
            pl.BlockSpec((1, D), lambda ph, i: (0, 0)),
        ],
        out_specs=pl.BlockSpec((RB, D), lambda ph, i: (i, 0)),
        out_shape=jax.ShapeDtypeStruct((N, D), jnp.float32),
        scratch_shapes=[
            pltpu.VMEM((N, D), jnp.float32),
            pltpu.VMEM((1, D), jnp.float32),
            pltpu.VMEM((1, D), jnp.float32),
        ],
    )(afull, afull, x, dis2, invd2, w, gamma, beta)


def kernel(x, edge_index, W, bn_gamma, bn_beta):
    ei = edge_index.astype(jnp.int32)
    src4 = ei[0].reshape(NS, NCH, CH, EB)
    dst4 = ei[1].reshape(NS, NCH, CH, EB)

    degp = _deg_kernel(dst4).reshape(NC, NPAD)
    degp_t = jnp.transpose(degp)[:N]
    xt, dis2, invd2 = _prep(degp_t, x)
    xt2 = xt.reshape(2 * N, H)
    afull = _agg_kernel(src4, dst4, xt2)
    return _mmbn(afull, x, dis2, invd2, W,
                 bn_gamma.reshape(1, D), bn_beta.reshape(1, D))


# speedup vs baseline: 23.0328x; 1.0351x over previous
import functools

import jax
import jax.numpy as jnp
from jax import lax
from jax.experimental import pallas as pl
from jax.experimental.pallas import tpu as pltpu
from jax.experimental.pallas import tpu_sc as plsc

N = 10000
E = 160000
D = 256
H = 128
NC, NS = 2, 16
NPAD = 10240
EB = 80
ERows = E // EB
RPT = ERows // NS
CH = 25
NCH = RPT // CH
RB = 1000
NBLK = N // RB
EPS_BN = 1e-5

_sc_mesh = plsc.VectorSubcoreMesh(core_axis_name="c", subcore_axis_name="s")


@functools.partial(
    pl.kernel,
    out_type=jax.ShapeDtypeStruct((NC * NPAD,), jnp.float32),
    mesh=_sc_mesh,
    scratch_types=[
        pltpu.VMEM((NCH, CH, EB), jnp.int32),
        pltpu.VMEM((EB,), jnp.float32),
        pltpu.VMEM((640,), jnp.float32),
        pltpu.VMEM_SHARED((NPAD,), jnp.float32),
        pltpu.SemaphoreType.DMA,
    ],
)
def _deg_kernel(dst4_hbm, out_hbm, idx_v, ones_v, zeros_v, deg_sh, sem):
    c = lax.axis_index("c")
    s = lax.axis_index("s")

    def fill_ones(i, carry):
        ones_v[pl.ds(i * 16, 16)] = jnp.full((16,), 1.0, jnp.float32)
        return carry

    lax.fori_loop(0, EB // 16, fill_ones, 0)

    def fill_zeros(i, carry):
        zeros_v[pl.ds(i * 16, 16)] = jnp.zeros((16,), jnp.float32)
        return carry

    lax.fori_loop(0, 640 // 16, fill_zeros, 0)
    pltpu.sync_copy(zeros_v, deg_sh.at[pl.ds(s * 640, 640)])
    plsc.subcore_barrier()

    pltpu.sync_copy(dst4_hbm.at[s], idx_v)

    def scat(b, carry):
        pltpu.async_copy(ones_v, deg_sh.at[idx_v.at[b // CH, b % CH]],
                         sem, add=True)
        return carry

    lax.fori_loop(0, RPT, scat, 0)

    def drain(b, carry):
        pltpu.make_async_copy(ones_v, deg_sh.at[pl.ds(0, EB)], sem).wait()
        return carry

    lax.fori_loop(0, RPT, drain, 0)
    plsc.subcore_barrier()
    pltpu.sync_copy(deg_sh.at[pl.ds(s * 640, 640)],
                    out_hbm.at[pl.ds(c * NPAD + s * 640, 640)])


@functools.partial(
    pl.kernel,
    out_type=jax.ShapeDtypeStruct((NC, NPAD, H), jnp.float32),
    mesh=_sc_mesh,
    scratch_types=[
        pltpu.VMEM((CH, EB), jnp.int32),
        pltpu.VMEM((CH, EB), jnp.int32),
        pltpu.VMEM((3, EB, H), jnp.float32),
        pltpu.VMEM_SHARED((NPAD, H), jnp.float32),
        pltpu.SemaphoreType.DMA,
        pltpu.SemaphoreType.DMA,
    ],
)
def _agg_kernel(src4_hbm, dst4_hbm, xt2_hbm, out_hbm,
                sidx_v, didx_v, rows_v, acc_sh, gsem, ssem):
    c = lax.axis_index("c")
    s = lax.axis_index("s")

    def fill_zeros(i, carry):
        rows_v[0, i // 8, pl.ds((i % 8) * 16, 16)] = jnp.zeros((16,),
                                                              jnp.float32)
        return carry

    lax.fori_loop(0, EB * H // 16, fill_zeros, 0)

    def zero_acc(j, carry):
        pltpu.sync_copy(rows_v.at[0], acc_sh.at[pl.ds(s * 640 + j * EB, EB)])
        return carry

    lax.fori_loop(0, 8, zero_acc, 0)
    plsc.subcore_barrier()

    def drain_gather(par):
        pltpu.make_async_copy(xt2_hbm.at[pl.ds(0, EB)], rows_v.at[par],
                              gsem).wait()

    def drain_scatter():
        pltpu.make_async_copy(rows_v.at[0], acc_sh.at[pl.ds(0, EB)],
                              ssem).wait()

    def chunk(ch, carry):
        pltpu.sync_copy(src4_hbm.at[s, ch], sidx_v)
        pltpu.sync_copy(dst4_hbm.at[s, ch], didx_v)

        def scale_idx(i, carry2):
            r = i // (EB // 16)
            l = i % (EB // 16)
            v = sidx_v[r, pl.ds(l * 16, 16)]
            sidx_v[r, pl.ds(l * 16, 16)] = v + c * N
            return carry2

        lax.fori_loop(0, CH * (EB // 16), scale_idx, 0)

        pltpu.async_copy(xt2_hbm.at[sidx_v.at[0]], rows_v.at[0], gsem)
        pltpu.async_copy(xt2_hbm.at[sidx_v.at[1]], rows_v.at[1], gsem)

        def batch(b, carry2):
            cur = lax.rem(b, 3)

            @pl.when(b + 2 < CH)
            def _():
                @pl.when(b >= 1)
                def _():
                    drain_scatter()
                pltpu.async_copy(xt2_hbm.at[sidx_v.at[b + 2]],
                                 rows_v.at[lax.rem(b + 2, 3)], gsem)

            drain_gather(cur)
            pltpu.async_copy(rows_v.at[cur], acc_sh.at[didx_v.at[b]],
                             ssem, add=True)
            return carry2

        lax.fori_loop(0, CH, batch, 0)
        drain_scatter()
        drain_scatter()
        drain_scatter()
        return carry

    lax.fori_loop(0, NCH, chunk, 0)
    plsc.subcore_barrier()
    pltpu.sync_copy(acc_sh.at[pl.ds(s * 640, 640)],
                    out_hbm.at[c, pl.ds(s * 640, 640)])


def _prep_body(degp_ref, x_ref, xt_ref, dis_ref, invd_ref):
    d = (degp_ref[:, 0:1] + degp_ref[:, 1:2]) * 0.5 + 1.0
    dis = lax.rsqrt(d)
    xt_ref[...] = x_ref[...] * dis
    dis_ref[...] = dis
    invd_ref[...] = 1.0 / d


def _prep(degp_t, x):
    return pl.pallas_call(
        _prep_body,
        grid=(NBLK, 2),
        in_specs=[
            pl.BlockSpec((RB, NC), lambda i, j: (i, 0)),
            pl.BlockSpec((RB, H), lambda i, j: (i, j)),
        ],
        out_specs=[
            pl.BlockSpec((RB, H), lambda i, j: (j * NBLK + i, 0)),
            pl.BlockSpec((RB, 1), lambda i, j: (i, 0)),
            pl.BlockSpec((RB, 1), lambda i, j: (i, 0)),
        ],
        out_shape=[
            jax.ShapeDtypeStruct((2 * N, H), jnp.float32),
            jax.ShapeDtypeStruct((N, 1), jnp.float32),
            jax.ShapeDtypeStruct((N, 1), jnp.float32),
        ],
    )(degp_t, x)


def _mmbn_body(al_ref, ar_ref, x_ref, dis_ref, invd_ref, w_ref, g_ref, b_ref,
               o_ref, p_scr, s1_scr, s2_scr):
    ph = pl.program_id(0)
    i = pl.program_id(1)

    @pl.when(ph == 0)
    def _():
        a = jnp.concatenate([al_ref[0], ar_ref[0]], axis=1)
        bm = a * dis_ref[...] + x_ref[...] * invd_ref[...]
        p = jnp.dot(bm, w_ref[...], preferred_element_type=jnp.float32)
        p_scr[pl.ds(i * RB, RB), :] = p
        s1 = jnp.sum(p, axis=0, keepdims=True)
        s2 = jnp.sum(p * p, axis=0, keepdims=True)

        @pl.when(i == 0)
        def _():
            s1_scr[...] = s1
            s2_scr[...] = s2

        @pl.when(i != 0)
        def _():
            s1_scr[...] += s1
            s2_scr[...] += s2

    @pl.when(ph == 1)
    def _():
        mean = s1_scr[...] * (1.0 / N)
        var = s2_scr[...] * (1.0 / N) - mean * mean
        scale = g_ref[...] * lax.rsqrt(var + EPS_BN)
        shift = b_ref[...] - mean * scale
        p = p_scr[pl.ds(i * RB, RB), :]
        o_ref[...] = jnp.maximum(p * scale + shift, 0.0)


def _mmbn(afull, x, dis2, invd2, w, gamma, beta):
    def only0(f):
        return lambda ph, i: f(jnp.where(ph == 0, i, NBLK - 1))

    return pl.pallas_call(
        _mmbn_body,
        grid=(2, NBLK),
        in_specs=[
            pl.BlockSpec((1, RB, H), only0(lambda i: (0, i, 0))),
            pl.BlockSpec((1, RB, H), only0(lambda i: (1, i, 0))),
            pl.BlockSpec((RB, D), only0(lambda i: (i, 0))),
            pl.BlockSpec((RB, 1), only0(lambda i: (i, 0))),
            pl.BlockSpec((RB, 1), only0(lambda i: (i, 0))),
            pl.BlockSpec((D, D), lambda ph, i: (0, 0)),
            pl.BlockSpec((1, D), lambda ph, i: (0, 0)),
            pl.BlockSpec((1, D), lambda ph, i: (0, 0)),
        ],
        out_specs=pl.BlockSpec(
            (RB, D), lambda ph, i: (jnp.where(ph == 0, 0, i), 0)),
        out_shape=jax.ShapeDtypeStruct((N, D), jnp.float32),
        scratch_shapes=[
            pltpu.VMEM((N, D), jnp.float32),
            pltpu.VMEM((1, D), jnp.float32),
            pltpu.VMEM((1, D), jnp.float32),
        ],
    )(afull, afull, x, dis2, invd2, w, gamma, beta)


def kernel(x, edge_index, W, bn_gamma, bn_beta):
    ei = edge_index.astype(jnp.int32)
    src4 = ei[0].reshape(NS, NCH, CH, EB)
    dst4 = ei[1].reshape(NS, NCH, CH, EB)

    degp = _deg_kernel(dst4).reshape(NC, NPAD)
    degp_t = jnp.transpose(degp)[:N]
    xt2, dis2, invd2 = _prep(degp_t, x)
    afull = _agg_kernel(src4, dst4, xt2)
    return _mmbn(afull, x, dis2, invd2, W,
                 bn_gamma.reshape(1, D), bn_beta.reshape(1, D))


# speedup vs baseline: 24.9314x; 1.0824x over previous
import functools

import jax
import jax.numpy as jnp
from jax import lax
from jax.experimental import pallas as pl
from jax.experimental.pallas import tpu as pltpu
from jax.experimental.pallas import tpu_sc as plsc

N = 10000
E = 160000
D = 256
H = 128
NC, NS = 2, 16
NPAD = 10240
EB = 80
ERows = E // EB
RPT = ERows // NS
CH = 25
NCH = RPT // CH
RB = 2000
NBLK = N // RB
EPS_BN = 1e-5

_sc_mesh = plsc.VectorSubcoreMesh(core_axis_name="c", subcore_axis_name="s")


@functools.partial(
    pl.kernel,
    out_type=jax.ShapeDtypeStruct((NC * NPAD,), jnp.float32),
    mesh=_sc_mesh,
    scratch_types=[
        pltpu.VMEM((NCH, CH, EB), jnp.int32),
        pltpu.VMEM((EB,), jnp.float32),
        pltpu.VMEM((640,), jnp.float32),
        pltpu.VMEM_SHARED((NPAD,), jnp.float32),
        pltpu.SemaphoreType.DMA,
    ],
)
def _deg_kernel(e4_hbm, out_hbm, idx_v, ones_v, zeros_v, deg_sh, sem):
    c = lax.axis_index("c")
    s = lax.axis_index("s")

    def fill_ones(i, carry):
        ones_v[pl.ds(i * 16, 16)] = jnp.full((16,), 1.0, jnp.float32)
        return carry

    lax.fori_loop(0, EB // 16, fill_ones, 0)

    def fill_zeros(i, carry):
        zeros_v[pl.ds(i * 16, 16)] = jnp.zeros((16,), jnp.float32)
        return carry

    lax.fori_loop(0, 640 // 16, fill_zeros, 0)
    pltpu.sync_copy(zeros_v, deg_sh.at[pl.ds(s * 640, 640)])
    plsc.subcore_barrier()

    pltpu.sync_copy(e4_hbm.at[1, pl.ds(s * NCH, NCH)], idx_v)

    def scat(b, carry):
        pltpu.async_copy(ones_v, deg_sh.at[idx_v.at[b // CH, b % CH]],
                         sem, add=True)
        return carry

    lax.fori_loop(0, RPT, scat, 0)

    def drain(b, carry):
        pltpu.make_async_copy(ones_v, deg_sh.at[pl.ds(0, EB)], sem).wait()
        return carry

    lax.fori_loop(0, RPT, drain, 0)
    plsc.subcore_barrier()
    pltpu.sync_copy(deg_sh.at[pl.ds(s * 640, 640)],
                    out_hbm.at[pl.ds(c * NPAD + s * 640, 640)])


@functools.partial(
    pl.kernel,
    out_type=jax.ShapeDtypeStruct((NC, NPAD, H), jnp.float32),
    mesh=_sc_mesh,
    scratch_types=[
        pltpu.VMEM((CH, EB), jnp.int32),
        pltpu.VMEM((CH, EB), jnp.int32),
        pltpu.VMEM((3, EB, H), jnp.float32),
        pltpu.VMEM_SHARED((NPAD, H), jnp.float32),
        pltpu.SemaphoreType.DMA,
        pltpu.SemaphoreType.DMA,
    ],
)
def _agg_kernel(e4_hbm, xt2_hbm, out_hbm,
                sidx_v, didx_v, rows_v, acc_sh, gsem, ssem):
    c = lax.axis_index("c")
    s = lax.axis_index("s")

    def fill_zeros(i, carry):
        rows_v[0, i // 8, pl.ds((i % 8) * 16, 16)] = jnp.zeros((16,),
                                                              jnp.float32)
        return carry

    lax.fori_loop(0, EB * H // 16, fill_zeros, 0)

    def zero_acc(j, carry):
        pltpu.sync_copy(rows_v.at[0], acc_sh.at[pl.ds(s * 640 + j * EB, EB)])
        return carry

    lax.fori_loop(0, 8, zero_acc, 0)
    plsc.subcore_barrier()

    def drain_gather(par):
        pltpu.make_async_copy(xt2_hbm.at[pl.ds(0, EB)], rows_v.at[par],
                              gsem).wait()

    def drain_scatter():
        pltpu.make_async_copy(rows_v.at[0], acc_sh.at[pl.ds(0, EB)],
                              ssem).wait()

    def chunk(ch, carry):
        pltpu.sync_copy(e4_hbm.at[0, s * NCH + ch], sidx_v)
        pltpu.sync_copy(e4_hbm.at[1, s * NCH + ch], didx_v)

        def scale_idx(i, carry2):
            r = i // (EB // 16)
            l = i % (EB // 16)
            v = sidx_v[r, pl.ds(l * 16, 16)]
            sidx_v[r, pl.ds(l * 16, 16)] = v + c * N
            return carry2

        lax.fori_loop(0, CH * (EB // 16), scale_idx, 0)

        pltpu.async_copy(xt2_hbm.at[sidx_v.at[0]], rows_v.at[0], gsem)
        pltpu.async_copy(xt2_hbm.at[sidx_v.at[1]], rows_v.at[1], gsem)

        def batch(b, carry2):
            cur = lax.rem(b, 3)

            @pl.when(b + 2 < CH)
            def _():
                @pl.when(b >= 1)
                def _():
                    drain_scatter()
                pltpu.async_copy(xt2_hbm.at[sidx_v.at[b + 2]],
                                 rows_v.at[lax.rem(b + 2, 3)], gsem)

            drain_gather(cur)
            pltpu.async_copy(rows_v.at[cur], acc_sh.at[didx_v.at[b]],
                             ssem, add=True)
            return carry2

        lax.fori_loop(0, CH, batch, 0)
        drain_scatter()
        drain_scatter()
        drain_scatter()
        return carry

    lax.fori_loop(0, NCH, chunk, 0)
    plsc.subcore_barrier()
    pltpu.sync_copy(acc_sh.at[pl.ds(s * 640, 640)],
                    out_hbm.at[c, pl.ds(s * 640, 640)])


def _prep_body(degp_ref, x_ref, xt_ref, dis_ref, invd_ref):
    d = (degp_ref[:, 0:1] + degp_ref[:, 1:2]) * 0.5 + 1.0
    dis = lax.rsqrt(d)
    xt_ref[...] = x_ref[...] * dis
    dis_ref[...] = dis
    invd_ref[...] = 1.0 / d


def _prep(degp_t, x):
    return pl.pallas_call(
        _prep_body,
        grid=(NBLK, 2),
        in_specs=[
            pl.BlockSpec((RB, NC), lambda i, j: (i, 0)),
            pl.BlockSpec((RB, H), lambda i, j: (i, j)),
        ],
        out_specs=[
            pl.BlockSpec((RB, H), lambda i, j: (j * NBLK + i, 0)),
            pl.BlockSpec((RB, 1), lambda i, j: (i, 0)),
            pl.BlockSpec((RB, 1), lambda i, j: (i, 0)),
        ],
        out_shape=[
            jax.ShapeDtypeStruct((2 * N, H), jnp.float32),
            jax.ShapeDtypeStruct((N, 1), jnp.float32),
            jax.ShapeDtypeStruct((N, 1), jnp.float32),
        ],
    )(degp_t, x)


def _mmbn_body(al_ref, ar_ref, x_ref, dis_ref, invd_ref, w_ref, g_ref, b_ref,
               o_ref, p_scr, s1_scr, s2_scr):
    ph = pl.program_id(0)
    i = pl.program_id(1)

    @pl.when(ph == 0)
    def _():
        a = jnp.concatenate([al_ref[0], ar_ref[0]], axis=1)
        bm = a * dis_ref[...] + x_ref[...] * invd_ref[...]
        p = jnp.dot(bm, w_ref[...], preferred_element_type=jnp.float32)
        p_scr[pl.ds(i * RB, RB), :] = p
        s1 = jnp.sum(p, axis=0, keepdims=True)
        s2 = jnp.sum(p * p, axis=0, keepdims=True)

        @pl.when(i == 0)
        def _():
            s1_scr[...] = s1
            s2_scr[...] = s2

        @pl.when(i != 0)
        def _():
            s1_scr[...] += s1
            s2_scr[...] += s2

    @pl.when(ph == 1)
    def _():
        mean = s1_scr[...] * (1.0 / N)
        var = s2_scr[...] * (1.0 / N) - mean * mean
        scale = g_ref[...] * lax.rsqrt(var + EPS_BN)
        shift = b_ref[...] - mean * scale
        p = p_scr[pl.ds(i * RB, RB), :]
        o_ref[...] = jnp.maximum(p * scale + shift, 0.0)


def _mmbn(afull, x, dis2, invd2, w, gamma, beta):
    def only0(f):
        return lambda ph, i: f(jnp.where(ph == 0, i, NBLK - 1))

    return pl.pallas_call(
        _mmbn_body,
        grid=(2, NBLK),
        in_specs=[
            pl.BlockSpec((1, RB, H), only0(lambda i: (0, i, 0))),
            pl.BlockSpec((1, RB, H), only0(lambda i: (1, i, 0))),
            pl.BlockSpec((RB, D), only0(lambda i: (i, 0))),
            pl.BlockSpec((RB, 1), only0(lambda i: (i, 0))),
            pl.BlockSpec((RB, 1), only0(lambda i: (i, 0))),
            pl.BlockSpec((D, D), lambda ph, i: (0, 0)),
            pl.BlockSpec((1, D), lambda ph, i: (0, 0)),
            pl.BlockSpec((1, D), lambda ph, i: (0, 0)),
        ],
        out_specs=pl.BlockSpec(
            (RB, D), lambda ph, i: (jnp.where(ph == 0, 0, i), 0)),
        out_shape=jax.ShapeDtypeStruct((N, D), jnp.float32),
        scratch_shapes=[
            pltpu.VMEM((N, D), jnp.float32),
            pltpu.VMEM((1, D), jnp.float32),
            pltpu.VMEM((1, D), jnp.float32),
        ],
    )(afull, afull, x, dis2, invd2, w, gamma, beta)


def kernel(x, edge_index, W, bn_gamma, bn_beta):
    e4 = edge_index.astype(jnp.int32).reshape(2, NS * NCH, CH, EB)

    degp = _deg_kernel(e4).reshape(NC, NPAD)
    degp_t = jnp.transpose(degp)[:N]
    xt2, dis2, invd2 = _prep(degp_t, x)
    afull = _agg_kernel(e4, xt2)
    return _mmbn(afull, x, dis2, invd2, W,
                 bn_gamma.reshape(1, D), bn_beta.reshape(1, D))


# speedup vs baseline: 26.7188x; 1.0717x over previous
import functools

import jax
import jax.numpy as jnp
from jax import lax
from jax.experimental import pallas as pl
from jax.experimental.pallas import tpu as pltpu
from jax.experimental.pallas import tpu_sc as plsc

N = 10000
E = 160000
D = 256
H = 128
NC, NS = 2, 16
NPAD = 10240
EB = 80
ERows = E // EB
RPT = ERows // NS
CH = 5
NCH = RPT // CH
RB = 2000
NBLK = N // RB
EPS_BN = 1e-5

_sc_mesh = plsc.VectorSubcoreMesh(core_axis_name="c", subcore_axis_name="s")


@functools.partial(
    pl.kernel,
    out_type=jax.ShapeDtypeStruct((NC * NPAD,), jnp.float32),
    mesh=_sc_mesh,
    scratch_types=[
        pltpu.VMEM((NCH, CH, EB), jnp.int32),
        pltpu.VMEM((EB,), jnp.float32),
        pltpu.VMEM((640,), jnp.float32),
        pltpu.VMEM_SHARED((NPAD,), jnp.float32),
        pltpu.SemaphoreType.DMA,
    ],
)
def _deg_kernel(e4_hbm, out_hbm, idx_v, ones_v, zeros_v, deg_sh, sem):
    c = lax.axis_index("c")
    s = lax.axis_index("s")

    def fill_ones(i, carry):
        ones_v[pl.ds(i * 16, 16)] = jnp.full((16,), 1.0, jnp.float32)
        return carry

    lax.fori_loop(0, EB // 16, fill_ones, 0)

    def fill_zeros(i, carry):
        zeros_v[pl.ds(i * 16, 16)] = jnp.zeros((16,), jnp.float32)
        return carry

    lax.fori_loop(0, 640 // 16, fill_zeros, 0)
    pltpu.sync_copy(zeros_v, deg_sh.at[pl.ds(s * 640, 640)])
    plsc.subcore_barrier()

    pltpu.sync_copy(e4_hbm.at[1, pl.ds(s * NCH, NCH)], idx_v)

    def scat(b, carry):
        pltpu.async_copy(ones_v, deg_sh.at[idx_v.at[b // CH, b % CH]],
                         sem, add=True)
        return carry

    lax.fori_loop(0, RPT, scat, 0)

    def drain(b, carry):
        pltpu.make_async_copy(ones_v, deg_sh.at[pl.ds(0, EB)], sem).wait()
        return carry

    lax.fori_loop(0, RPT, drain, 0)
    plsc.subcore_barrier()
    pltpu.sync_copy(deg_sh.at[pl.ds(s * 640, 640)],
                    out_hbm.at[pl.ds(c * NPAD + s * 640, 640)])


@functools.partial(
    pl.kernel,
    out_type=jax.ShapeDtypeStruct((NC, NPAD, H), jnp.float32),
    mesh=_sc_mesh,
    scratch_types=[
        pltpu.VMEM((2, CH, EB), jnp.int32),
        pltpu.VMEM((2, CH, EB), jnp.int32),
        pltpu.VMEM((3, EB, H), jnp.float32),
        pltpu.VMEM_SHARED((NPAD, H), jnp.float32),
        pltpu.SemaphoreType.DMA,
        pltpu.SemaphoreType.DMA,
        pltpu.SemaphoreType.DMA,
    ],
)
def _agg_kernel(e4_hbm, xt2_hbm, out_hbm,
                sidx_v, didx_v, rows_v, acc_sh, gsem, ssem, isem):
    c = lax.axis_index("c")
    s = lax.axis_index("s")

    def fill_zeros(i, carry):
        rows_v[0, i // 8, pl.ds((i % 8) * 16, 16)] = jnp.zeros((16,),
                                                              jnp.float32)
        return carry

    lax.fori_loop(0, EB * H // 16, fill_zeros, 0)

    def zero_acc(j, carry):
        pltpu.sync_copy(rows_v.at[0], acc_sh.at[pl.ds(s * 640 + j * EB, EB)])
        return carry

    lax.fori_loop(0, 8, zero_acc, 0)
    plsc.subcore_barrier()

    def drain_gather(par):
        pltpu.make_async_copy(xt2_hbm.at[pl.ds(0, EB)], rows_v.at[par],
                              gsem).wait()

    def drain_scatter():
        pltpu.make_async_copy(rows_v.at[0], acc_sh.at[pl.ds(0, EB)],
                              ssem).wait()

    def transform(buf, carry):
        def scale_idx(i, carry2):
            r = i // (EB // 16)
            l = i % (EB // 16)
            v = sidx_v[buf, r, pl.ds(l * 16, 16)]
            sidx_v[buf, r, pl.ds(l * 16, 16)] = v + c * N
            return carry2

        return lax.fori_loop(0, CH * (EB // 16), scale_idx, carry)

    pltpu.sync_copy(e4_hbm.at[0, s * NCH], sidx_v.at[0])
    pltpu.sync_copy(e4_hbm.at[1, s * NCH], didx_v.at[0])
    transform(0, 0)
    pltpu.async_copy(xt2_hbm.at[sidx_v.at[0, 0]], rows_v.at[0], gsem)
    pltpu.async_copy(xt2_hbm.at[sidx_v.at[0, 1]], rows_v.at[1], gsem)

    def batch(b, carry):
        ch = b // CH
        r = b - ch * CH
        cur = lax.rem(ch, 2)
        nxt = 1 - cur

        @pl.when((r == 2) & (ch + 1 < NCH))
        def _():
            pltpu.async_copy(e4_hbm.at[0, s * NCH + ch + 1],
                             sidx_v.at[nxt], isem)
            pltpu.async_copy(e4_hbm.at[1, s * NCH + ch + 1],
                             didx_v.at[nxt], isem)

        @pl.when((r == 3) & (ch + 1 < NCH))
        def _():
            pltpu.make_async_copy(e4_hbm.at[0, 0], sidx_v.at[0], isem).wait()
            pltpu.make_async_copy(e4_hbm.at[0, 0], didx_v.at[0], isem).wait()
            transform(nxt, 0)

        @pl.when(b + 2 < RPT)
        def _():
            @pl.when(b >= 1)
            def _():
                drain_scatter()
            g = b + 2
            gch = g // CH
            pltpu.async_copy(
                xt2_hbm.at[sidx_v.at[lax.rem(gch, 2), g - gch * CH]],
                rows_v.at[lax.rem(g, 3)], gsem)

        drain_gather(lax.rem(b, 3))
        pltpu.async_copy(rows_v.at[lax.rem(b, 3)], acc_sh.at[didx_v.at[cur, r]],
                         ssem, add=True)
        return carry

    lax.fori_loop(0, RPT, batch, 0)
    drain_scatter()
    drain_scatter()
    drain_scatter()
    plsc.subcore_barrier()
    pltpu.sync_copy(acc_sh.at[pl.ds(s * 640, 640)],
                    out_hbm.at[c, pl.ds(s * 640, 640)])


def _prep_body(degp_ref, x_ref, xt_ref, dis_ref, invd_ref):
    d = (degp_ref[:, 0:1] + degp_ref[:, 1:2]) * 0.5 + 1.0
    dis = lax.rsqrt(d)
    xt_ref[...] = x_ref[...] * dis
    dis_ref[...] = dis
    invd_ref[...] = 1.0 / d


def _prep(degp_t, x):
    return pl.pallas_call(
        _prep_body,
        grid=(NBLK, 2),
        in_specs=[
            pl.BlockSpec((RB, NC), lambda i, j: (i, 0)),
            pl.BlockSpec((RB, H), lambda i, j: (i, j)),
        ],
        out_specs=[
            pl.BlockSpec((RB, H), lambda i, j: (j * NBLK + i, 0)),
            pl.BlockSpec((RB, 1), lambda i, j: (i, 0)),
            pl.BlockSpec((RB, 1), lambda i, j: (i, 0)),
        ],
        out_shape=[
            jax.ShapeDtypeStruct((2 * N, H), jnp.float32),
            jax.ShapeDtypeStruct((N, 1), jnp.float32),
            jax.ShapeDtypeStruct((N, 1), jnp.float32),
        ],
    )(degp_t, x)


def _mmbn_body(al_ref, ar_ref, x_ref, dis_ref, invd_ref, w_ref, g_ref, b_ref,
               o_ref, p_scr, s1_scr, s2_scr):
    ph = pl.program_id(0)
    i = pl.program_id(1)

    @pl.when(ph == 0)
    def _():
        a = jnp.concatenate([al_ref[0], ar_ref[0]], axis=1)
        bm = a * dis_ref[...] + x_ref[...] * invd_ref[...]
        p = jnp.dot(bm, w_ref[...], preferred_element_type=jnp.float32)
        p_scr[pl.ds(i * RB, RB), :] = p
        s1 = jnp.sum(p, axis=0, keepdims=True)
        s2 = jnp.sum(p * p, axis=0, keepdims=True)

        @pl.when(i == 0)
        def _():
            s1_scr[...] = s1
            s2_scr[...] = s2

        @pl.when(i != 0)
        def _():
            s1_scr[...] += s1
            s2_scr[...] += s2

    @pl.when(ph == 1)
    def _():
        mean = s1_scr[...] * (1.0 / N)
        var = s2_scr[...] * (1.0 / N) - mean * mean
        scale = g_ref[...] * lax.rsqrt(var + EPS_BN)
        shift = b_ref[...] - mean * scale
        p = p_scr[pl.ds(i * RB, RB), :]
        o_ref[...] = jnp.maximum(p * scale + shift, 0.0)


def _mmbn(afull, x, dis2, invd2, w, gamma, beta):
    def only0(f):
        return lambda ph, i: f(jnp.where(ph == 0, i, NBLK - 1))

    return pl.pallas_call(
        _mmbn_body,
        grid=(2, NBLK),
        in_specs=[
            pl.BlockSpec((1, RB, H), only0(lambda i: (0, i, 0))),
            pl.BlockSpec((1, RB, H), only0(lambda i: (1, i, 0))),
            pl.BlockSpec((RB, D), only0(lambda i: (i, 0))),
            pl.BlockSpec((RB, 1), only0(lambda i: (i, 0))),
            pl.BlockSpec((RB, 1), only0(lambda i: (i, 0))),
            pl.BlockSpec((D, D), lambda ph, i: (0, 0)),
            pl.BlockSpec((1, D), lambda ph, i: (0, 0)),
            pl.BlockSpec((1, D), lambda ph, i: (0, 0)),
        ],
        out_specs=pl.BlockSpec(
            (RB, D), lambda ph, i: (jnp.where(ph == 0, 0, i), 0)),
        out_shape=jax.ShapeDtypeStruct((N, D), jnp.float32),
        scratch_shapes=[
            pltpu.VMEM((N, D), jnp.float32),
            pltpu.VMEM((1, D), jnp.float32),
            pltpu.VMEM((1, D), jnp.float32),
        ],
    )(afull, afull, x, dis2, invd2, w, gamma, beta)


def kernel(x, edge_index, W, bn_gamma, bn_beta):
    e4 = edge_index.astype(jnp.int32).reshape(2, NS * NCH, CH, EB)

    degp = _deg_kernel(e4).reshape(NC, NPAD)
    degp_t = jnp.transpose(degp)[:N]
    xt2, dis2, invd2 = _prep(degp_t, x)
    afull = _agg_kernel(e4, xt2)
    return _mmbn(afull, x, dis2, invd2, W,
                 bn_gamma.reshape(1, D), bn_beta.reshape(1, D))


# speedup vs baseline: 27.5669x; 1.0317x over previous
import functools

import jax
import jax.numpy as jnp
from jax import lax
from jax.experimental import pallas as pl
from jax.experimental.pallas import tpu as pltpu
from jax.experimental.pallas import tpu_sc as plsc

N = 10000
E = 160000
D = 256
H = 128
NC, NS = 2, 16
NPAD = 10240
EB = 80
ERows = E // EB
RPT = ERows // NS
CH = 5
NCH = RPT // CH
RB = 2000
NBLK = N // RB
EPS_BN = 1e-5

_sc_mesh = plsc.VectorSubcoreMesh(core_axis_name="c", subcore_axis_name="s")


@functools.partial(
    pl.kernel,
    out_type=jax.ShapeDtypeStruct((NC * NPAD,), jnp.float32),
    mesh=_sc_mesh,
    scratch_types=[
        pltpu.VMEM((NCH, CH, EB), jnp.int32),
        pltpu.VMEM((EB,), jnp.float32),
        pltpu.VMEM((640,), jnp.float32),
        pltpu.VMEM_SHARED((NPAD,), jnp.float32),
        pltpu.SemaphoreType.DMA,
    ],
)
def _deg_kernel(e4_hbm, out_hbm, idx_v, ones_v, zeros_v, deg_sh, sem):
    c = lax.axis_index("c")
    s = lax.axis_index("s")

    def fill_ones(i, carry):
        ones_v[pl.ds(i * 16, 16)] = jnp.full((16,), 1.0, jnp.float32)
        return carry

    lax.fori_loop(0, EB // 16, fill_ones, 0)

    def fill_zeros(i, carry):
        zeros_v[pl.ds(i * 16, 16)] = jnp.zeros((16,), jnp.float32)
        return carry

    lax.fori_loop(0, 640 // 16, fill_zeros, 0)
    pltpu.sync_copy(zeros_v, deg_sh.at[pl.ds(s * 640, 640)])
    plsc.subcore_barrier()

    pltpu.sync_copy(e4_hbm.at[1, pl.ds(s * NCH, NCH)], idx_v)

    def scat(b, carry):
        pltpu.async_copy(ones_v, deg_sh.at[idx_v.at[b // CH, b % CH]],
                         sem, add=True)
        return carry

    lax.fori_loop(0, RPT, scat, 0)

    def drain(b, carry):
        pltpu.make_async_copy(ones_v, deg_sh.at[pl.ds(0, EB)], sem).wait()
        return carry

    lax.fori_loop(0, RPT, drain, 0)
    plsc.subcore_barrier()
    pltpu.sync_copy(deg_sh.at[pl.ds(s * 640, 640)],
                    out_hbm.at[pl.ds(c * NPAD + s * 640, 640)])


@functools.partial(
    pl.kernel,
    out_type=jax.ShapeDtypeStruct((NC, NPAD, H), jnp.float32),
    mesh=_sc_mesh,
    scratch_types=[
        pltpu.VMEM((2, CH, EB), jnp.int32),
        pltpu.VMEM((2, CH, EB), jnp.int32),
        pltpu.VMEM((3, EB, H), jnp.float32),
        pltpu.VMEM_SHARED((NPAD, H), jnp.float32),
        pltpu.SemaphoreType.DMA,
        pltpu.SemaphoreType.DMA,
        pltpu.SemaphoreType.DMA,
    ],
)
def _agg_kernel(e4_hbm, xt2_hbm, out_hbm,
                sidx_v, didx_v, rows_v, acc_sh, gsem, ssem, isem):
    c = lax.axis_index("c")
    s = lax.axis_index("s")

    def fill_zeros(i, carry):
        rows_v[2, i // 8, pl.ds((i % 8) * 16, 16)] = jnp.zeros((16,),
                                                              jnp.float32)
        return carry

    lax.fori_loop(0, EB * H // 16, fill_zeros, 0)

    def zero_acc(j, carry):
        pltpu.async_copy(rows_v.at[2],
                         acc_sh.at[pl.ds(s * 640 + j * EB, EB)], isem)
        return carry

    lax.fori_loop(0, 8, zero_acc, 0)

    def drain_gather(par):
        pltpu.make_async_copy(xt2_hbm.at[pl.ds(0, EB)], rows_v.at[par],
                              gsem).wait()

    def drain_scatter():
        pltpu.make_async_copy(rows_v.at[0], acc_sh.at[pl.ds(0, EB)],
                              ssem).wait()

    def transform(buf, carry):
        def scale_idx(i, carry2):
            r = i // (EB // 16)
            l = i % (EB // 16)
            v = sidx_v[buf, r, pl.ds(l * 16, 16)]
            sidx_v[buf, r, pl.ds(l * 16, 16)] = v + c * N
            return carry2

        return lax.fori_loop(0, CH * (EB // 16), scale_idx, carry)

    pltpu.sync_copy(e4_hbm.at[0, s * NCH], sidx_v.at[0])
    pltpu.sync_copy(e4_hbm.at[1, s * NCH], didx_v.at[0])
    transform(0, 0)
    pltpu.async_copy(xt2_hbm.at[sidx_v.at[0, 0]], rows_v.at[0], gsem)
    pltpu.async_copy(xt2_hbm.at[sidx_v.at[0, 1]], rows_v.at[1], gsem)

    def zero_drain(j, carry):
        pltpu.make_async_copy(rows_v.at[2], acc_sh.at[pl.ds(0, EB)],
                              isem).wait()
        return carry

    lax.fori_loop(0, 8, zero_drain, 0)
    plsc.subcore_barrier()

    def batch(b, carry):
        ch = b // CH
        r = b - ch * CH
        cur = lax.rem(ch, 2)
        nxt = 1 - cur

        @pl.when((r == 2) & (ch + 1 < NCH))
        def _():
            pltpu.async_copy(e4_hbm.at[0, s * NCH + ch + 1],
                             sidx_v.at[nxt], isem)
            pltpu.async_copy(e4_hbm.at[1, s * NCH + ch + 1],
                             didx_v.at[nxt], isem)

        @pl.when((r == 3) & (ch + 1 < NCH))
        def _():
            pltpu.make_async_copy(e4_hbm.at[0, 0], sidx_v.at[0], isem).wait()
            pltpu.make_async_copy(e4_hbm.at[0, 0], didx_v.at[0], isem).wait()
            transform(nxt, 0)

        @pl.when(b + 2 < RPT)
        def _():
            @pl.when(b >= 1)
            def _():
                drain_scatter()
            g = b + 2
            gch = g // CH
            pltpu.async_copy(
                xt2_hbm.at[sidx_v.at[lax.rem(gch, 2), g - gch * CH]],
                rows_v.at[lax.rem(g, 3)], gsem)

        drain_gather(lax.rem(b, 3))
        pltpu.async_copy(rows_v.at[lax.rem(b, 3)], acc_sh.at[didx_v.at[cur, r]],
                         ssem, add=True)
        return carry

    lax.fori_loop(0, RPT, batch, 0)
    drain_scatter()
    drain_scatter()
    drain_scatter()
    plsc.subcore_barrier()
    pltpu.sync_copy(acc_sh.at[pl.ds(s * 640, 640)],
                    out_hbm.at[c, pl.ds(s * 640, 640)])


def _prep_body(degp_ref, x_ref, xt_ref, dis_ref, invd_ref):
    d = (degp_ref[:, 0:1] + degp_ref[:, 1:2]) * 0.5 + 1.0
    dis = lax.rsqrt(d)
    xt_ref[...] = x_ref[...] * dis
    dis_ref[...] = dis
    invd_ref[...] = 1.0 / d


def _prep(degp_t, x):
    return pl.pallas_call(
        _prep_body,
        grid=(2,),
        in_specs=[
            pl.BlockSpec((N, NC), lambda j: (0, 0)),
            pl.BlockSpec((N, H), lambda j: (0, j)),
        ],
        out_specs=[
            pl.BlockSpec((N, H), lambda j: (j, 0)),
            pl.BlockSpec((N, 1), lambda j: (0, 0)),
            pl.BlockSpec((N, 1), lambda j: (0, 0)),
        ],
        out_shape=[
            jax.ShapeDtypeStruct((2 * N, H), jnp.float32),
            jax.ShapeDtypeStruct((N, 1), jnp.float32),
            jax.ShapeDtypeStruct((N, 1), jnp.float32),
        ],
    )(degp_t, x)


def _mmbn_body(al_ref, ar_ref, x_ref, dis_ref, invd_ref, w_ref, g_ref, b_ref,
               o_ref, p_scr, s1_scr, s2_scr):
    ph = pl.program_id(0)
    i = pl.program_id(1)

    @pl.when(ph == 0)
    def _():
        a = jnp.concatenate([al_ref[0], ar_ref[0]], axis=1)
        bm = a * dis_ref[...] + x_ref[...] * invd_ref[...]
        p = jnp.dot(bm, w_ref[...], preferred_element_type=jnp.float32)
        p_scr[pl.ds(i * RB, RB), :] = p
        s1 = jnp.sum(p, axis=0, keepdims=True)
        s2 = jnp.sum(p * p, axis=0, keepdims=True)

        @pl.when(i == 0)
        def _():
            s1_scr[...] = s1
            s2_scr[...] = s2

        @pl.when(i != 0)
        def _():
            s1_scr[...] += s1
            s2_scr[...] += s2

    @pl.when(ph == 1)
    def _():
        mean = s1_scr[...] * (1.0 / N)
        var = s2_scr[...] * (1.0 / N) - mean * mean
        scale = g_ref[...] * lax.rsqrt(var + EPS_BN)
        shift = b_ref[...] - mean * scale
        p = p_scr[pl.ds(i * RB, RB), :]
        o_ref[...] = jnp.maximum(p * scale + shift, 0.0)


def _mmbn(afull, x, dis2, invd2, w, gamma, beta):
    def only0(f):
        return lambda ph, i: f(jnp.where(ph == 0, i, NBLK - 1))

    return pl.pallas_call(
        _mmbn_body,
        grid=(2, NBLK),
        in_specs=[
            pl.BlockSpec((1, RB, H), only0(lambda i: (0, i, 0))),
            pl.BlockSpec((1, RB, H), only0(lambda i: (1, i, 0))),
            pl.BlockSpec((RB, D), only0(lambda i: (i, 0))),
            pl.BlockSpec((RB, 1), only0(lambda i: (i, 0))),
            pl.BlockSpec((RB, 1), only0(lambda i: (i, 0))),
            pl.BlockSpec((D, D), lambda ph, i: (0, 0)),
            pl.BlockSpec((1, D), lambda ph, i: (0, 0)),
            pl.BlockSpec((1, D), lambda ph, i: (0, 0)),
        ],
        out_specs=pl.BlockSpec(
            (RB, D), lambda ph, i: (jnp.where(ph == 0, 0, i), 0)),
        out_shape=jax.ShapeDtypeStruct((N, D), jnp.float32),
        scratch_shapes=[
            pltpu.VMEM((N, D), jnp.float32),
            pltpu.VMEM((1, D), jnp.float32),
            pltpu.VMEM((1, D), jnp.float32),
        ],
    )(afull, afull, x, dis2, invd2, w, gamma, beta)


def kernel(x, edge_index, W, bn_gamma, bn_beta):
    e4 = edge_index.astype(jnp.int32).reshape(2, NS * NCH, CH, EB)

    degp = _deg_kernel(e4).reshape(NC, NPAD)
    degp_t = jnp.transpose(degp)[:N]
    xt2, dis2, invd2 = _prep(degp_t, x)
    afull = _agg_kernel(e4, xt2)
    return _mmbn(afull, x, dis2, invd2, W,
                 bn_gamma.reshape(1, D), bn_beta.reshape(1, D))


# speedup vs baseline: 28.8164x; 1.0453x over previous
import functools

import jax
import jax.numpy as jnp
from jax import lax
from jax.experimental import pallas as pl
from jax.experimental.pallas import tpu as pltpu
from jax.experimental.pallas import tpu_sc as plsc

N = 10000
E = 160000
D = 256
H = 128
NC, NS = 2, 16
NPAD = 10240
EB = 80
ERows = E // EB
RPT = ERows // NS
CH = 5
NCH = RPT // CH
RB = 2000
NBLK = N // RB
EPS_BN = 1e-5

_sc_mesh = plsc.VectorSubcoreMesh(core_axis_name="c", subcore_axis_name="s")


@functools.partial(
    pl.kernel,
    out_type=jax.ShapeDtypeStruct((NPAD,), jnp.float32),
    mesh=_sc_mesh,
    scratch_types=[
        pltpu.VMEM((NCH, CH, EB), jnp.int32),
        pltpu.VMEM((EB,), jnp.float32),
        pltpu.VMEM((640,), jnp.float32),
        pltpu.VMEM_SHARED((NPAD,), jnp.float32),
        pltpu.SemaphoreType.DMA,
    ],
)
def _deg_kernel(e4_hbm, out_hbm, idx_v, ones_v, zeros_v, deg_sh, sem):
    c = lax.axis_index("c")
    s = lax.axis_index("s")

    def fill_ones(i, carry):
        ones_v[pl.ds(i * 16, 16)] = jnp.full((16,), 1.0, jnp.float32)
        return carry

    lax.fori_loop(0, EB // 16, fill_ones, 0)

    def fill_zeros(i, carry):
        zeros_v[pl.ds(i * 16, 16)] = jnp.zeros((16,), jnp.float32)
        return carry

    lax.fori_loop(0, 640 // 16, fill_zeros, 0)
    pltpu.sync_copy(zeros_v, deg_sh.at[pl.ds(s * 640, 640)])
    plsc.subcore_barrier()

    pltpu.sync_copy(e4_hbm.at[1, pl.ds(s * NCH, NCH)], idx_v)

    def scat(b, carry):
        pltpu.async_copy(ones_v, deg_sh.at[idx_v.at[b // CH, b % CH]],
                         sem, add=True)
        return carry

    lax.fori_loop(0, RPT, scat, 0)

    def drain(b, carry):
        pltpu.make_async_copy(ones_v, deg_sh.at[pl.ds(0, EB)], sem).wait()
        return carry

    lax.fori_loop(0, RPT, drain, 0)
    plsc.subcore_barrier()

    @pl.when(c == 0)
    def _():
        pltpu.sync_copy(deg_sh.at[pl.ds(s * 640, 640)],
                        out_hbm.at[pl.ds(s * 640, 640)])


@functools.partial(
    pl.kernel,
    out_type=jax.ShapeDtypeStruct((NC, NPAD, H), jnp.float32),
    mesh=_sc_mesh,
    scratch_types=[
        pltpu.VMEM((2, CH, EB), jnp.int32),
        pltpu.VMEM((2, CH, EB), jnp.int32),
        pltpu.VMEM((3, EB, H), jnp.float32),
        pltpu.VMEM_SHARED((NPAD, H), jnp.float32),
        pltpu.SemaphoreType.DMA,
        pltpu.SemaphoreType.DMA,
        pltpu.SemaphoreType.DMA,
    ],
)
def _agg_kernel(e4_hbm, xt2_hbm, out_hbm,
                sidx_v, didx_v, rows_v, acc_sh, gsem, ssem, isem):
    c = lax.axis_index("c")
    s = lax.axis_index("s")

    nz = jnp.where(s == NS - 1, (N - (NS - 1) * 640) // EB, 640 // EB)

    def init_acc(j, carry):
        pltpu.async_copy(xt2_hbm.at[pl.ds(c * N + s * 640 + j * EB, EB)],
                         acc_sh.at[pl.ds(s * 640 + j * EB, EB)], isem)
        return carry

    lax.fori_loop(0, nz, init_acc, 0)

    def drain_gather(par):
        pltpu.make_async_copy(xt2_hbm.at[pl.ds(0, EB)], rows_v.at[par],
                              gsem).wait()

    def drain_scatter():
        pltpu.make_async_copy(rows_v.at[0], acc_sh.at[pl.ds(0, EB)],
                              ssem).wait()

    def transform(buf, carry):
        def scale_idx(i, carry2):
            r = i // (EB // 16)
            l = i % (EB // 16)
            v = sidx_v[buf, r, pl.ds(l * 16, 16)]
            sidx_v[buf, r, pl.ds(l * 16, 16)] = v + c * N
            return carry2

        return lax.fori_loop(0, CH * (EB // 16), scale_idx, carry)

    pltpu.sync_copy(e4_hbm.at[0, s * NCH], sidx_v.at[0])
    pltpu.sync_copy(e4_hbm.at[1, s * NCH], didx_v.at[0])
    transform(0, 0)
    pltpu.async_copy(xt2_hbm.at[sidx_v.at[0, 0]], rows_v.at[0], gsem)
    pltpu.async_copy(xt2_hbm.at[sidx_v.at[0, 1]], rows_v.at[1], gsem)

    def init_drain(j, carry):
        pltpu.make_async_copy(xt2_hbm.at[pl.ds(0, EB)],
                              acc_sh.at[pl.ds(0, EB)], isem).wait()
        return carry

    lax.fori_loop(0, nz, init_drain, 0)
    plsc.subcore_barrier()

    def batch(b, carry):
        ch = b // CH
        r = b - ch * CH
        cur = lax.rem(ch, 2)
        nxt = 1 - cur

        @pl.when((r == 2) & (ch + 1 < NCH))
        def _():
            pltpu.async_copy(e4_hbm.at[0, s * NCH + ch + 1],
                             sidx_v.at[nxt], isem)
            pltpu.async_copy(e4_hbm.at[1, s * NCH + ch + 1],
                             didx_v.at[nxt], isem)

        @pl.when((r == 3) & (ch + 1 < NCH))
        def _():
            pltpu.make_async_copy(e4_hbm.at[0, 0], sidx_v.at[0], isem).wait()
            pltpu.make_async_copy(e4_hbm.at[0, 0], didx_v.at[0], isem).wait()
            transform(nxt, 0)

        @pl.when(b + 2 < RPT)
        def _():
            @pl.when(b >= 1)
            def _():
                drain_scatter()
            g = b + 2
            gch = g // CH
            pltpu.async_copy(
                xt2_hbm.at[sidx_v.at[lax.rem(gch, 2), g - gch * CH]],
                rows_v.at[lax.rem(g, 3)], gsem)

        drain_gather(lax.rem(b, 3))
        pltpu.async_copy(rows_v.at[lax.rem(b, 3)], acc_sh.at[didx_v.at[cur, r]],
                         ssem, add=True)
        return carry

    lax.fori_loop(0, RPT, batch, 0)
    drain_scatter()
    drain_scatter()
    drain_scatter()
    plsc.subcore_barrier()
    pltpu.sync_copy(acc_sh.at[pl.ds(s * 640, 640)],
                    out_hbm.at[c, pl.ds(s * 640, 640)])


def _prep_body(degc_ref, x_ref, xt_ref, dis_ref):
    dis = lax.rsqrt(degc_ref[...] + 1.0)
    xt_ref[...] = x_ref[...] * dis
    dis_ref[...] = dis


def _prep(degc, x):
    return pl.pallas_call(
        _prep_body,
        grid=(2,),
        in_specs=[
            pl.BlockSpec((N, 1), lambda j: (0, 0)),
            pl.BlockSpec((N, H), lambda j: (0, j)),
        ],
        out_specs=[
            pl.BlockSpec((N, H), lambda j: (j, 0)),
            pl.BlockSpec((N, 1), lambda j: (0, 0)),
        ],
        out_shape=[
            jax.ShapeDtypeStruct((2 * N, H), jnp.float32),
            jax.ShapeDtypeStruct((N, 1), jnp.float32),
        ],
    )(degc, x)


def _mmbn_body(al_ref, ar_ref, dis_ref, w_ref, g_ref, b_ref,
               o_ref, p_scr, s1_scr, s2_scr):
    ph = pl.program_id(0)
    i = pl.program_id(1)

    @pl.when(ph == 0)
    def _():
        a = jnp.concatenate([al_ref[0], ar_ref[0]], axis=1)
        bm = a * dis_ref[...]
        p = jnp.dot(bm, w_ref[...], preferred_element_type=jnp.float32)
        p_scr[pl.ds(i * RB, RB), :] = p
        s1 = jnp.sum(p, axis=0, keepdims=True)
        s2 = jnp.sum(p * p, axis=0, keepdims=True)

        @pl.when(i == 0)
        def _():
            s1_scr[...] = s1
            s2_scr[...] = s2

        @pl.when(i != 0)
        def _():
            s1_scr[...] += s1
            s2_scr[...] += s2

    @pl.when(ph == 1)
    def _():
        mean = s1_scr[...] * (1.0 / N)
        var = s2_scr[...] * (1.0 / N) - mean * mean
        scale = g_ref[...] * lax.rsqrt(var + EPS_BN)
        shift = b_ref[...] - mean * scale
        p = p_scr[pl.ds(i * RB, RB), :]
        o_ref[...] = jnp.maximum(p * scale + shift, 0.0)


def _mmbn(afull, dis2, w, gamma, beta):
    def only0(f):
        return lambda ph, i: f(jnp.where(ph == 0, i, NBLK - 1))

    return pl.pallas_call(
        _mmbn_body,
        grid=(2, NBLK),
        in_specs=[
            pl.BlockSpec((1, RB, H), only0(lambda i: (0, i, 0))),
            pl.BlockSpec((1, RB, H), only0(lambda i: (1, i, 0))),
            pl.BlockSpec((RB, 1), only0(lambda i: (i, 0))),
            pl.BlockSpec((D, D), lambda ph, i: (0, 0)),
            pl.BlockSpec((1, D), lambda ph, i: (0, 0)),
            pl.BlockSpec((1, D), lambda ph, i: (0, 0)),
        ],
        out_specs=pl.BlockSpec(
            (RB, D), lambda ph, i: (jnp.where(ph == 0, 0, i), 0)),
        out_shape=jax.ShapeDtypeStruct((N, D), jnp.float32),
        scratch_shapes=[
            pltpu.VMEM((N, D), jnp.float32),
            pltpu.VMEM((1, D), jnp.float32),
            pltpu.VMEM((1, D), jnp.float32),
        ],
    )(afull, afull, dis2, w, gamma, beta)


def kernel(x, edge_index, W, bn_gamma, bn_beta):
    e4 = edge_index.astype(jnp.int32).reshape(2, NS * NCH, CH, EB)

    degc = _deg_kernel(e4).reshape(NPAD, 1)
    xt2, dis2 = _prep(degc, x)
    afull = _agg_kernel(e4, xt2)
    return _mmbn(afull, dis2, W,
                 bn_gamma.reshape(1, D), bn_beta.reshape(1, D))
